# Initial kernel scaffold; baseline (speedup 1.0000x reference)
import jax
import jax.numpy as jnp
from jax.experimental import pallas as pl


def kernel(feature_map, proposals, scores, W_fc, b_fc, W_l, b_l, W_s, b_s):
    raise NotImplementedError("write your pallas kernel here")


# speedup vs baseline: 18.5127x; 18.5127x over previous
import functools

import jax
import jax.numpy as jnp
from jax.experimental import pallas as pl
import jax.experimental.pallas.tpu as pltpu

_N = 5000
_NP = 5120
_TB = 256
_K = 500
_KP = 512
_C = 512
_H = 25
_W = 25
_IOU_THR = 0.7
_STRIDE = 32.0
_NUM_CLASS = 20
_BOX_BLK = 128
_ROWS_BLK = _BOX_BLK * 7
_KDIM = 25088
_FC = 4096


def _fiota(shape, dim):
    return jax.lax.broadcasted_iota(jnp.int32, shape, dim).astype(jnp.float32)


def _nms_kernel(boxes_ref, keep_ref):
    x1 = boxes_ref[0:1, :]
    y1 = boxes_ref[1:2, :]
    x2 = boxes_ref[2:3, :]
    y2 = boxes_ref[3:4, :]
    areas = (x2 - x1) * (y2 - y1)
    colg = _fiota((_TB, _NP), 1)

    keep = jnp.ones((1, _NP), jnp.float32)
    for b in range(_NP // _TB):
        s = b * _TB
        x1b = jnp.transpose(x1[:, s:s + _TB])
        y1b = jnp.transpose(y1[:, s:s + _TB])
        x2b = jnp.transpose(x2[:, s:s + _TB])
        y2b = jnp.transpose(y2[:, s:s + _TB])
        ab = jnp.transpose(areas[:, s:s + _TB])
        xx1 = jnp.maximum(x1b, x1)
        yy1 = jnp.maximum(y1b, y1)
        xx2 = jnp.minimum(x2b, x2)
        yy2 = jnp.minimum(y2b, y2)
        inter = jnp.maximum(xx2 - xx1, 0.0) * jnp.maximum(yy2 - yy1, 0.0)
        iou = inter / (ab + areas - inter + 1e-9)
        m = (iou > _IOU_THR).astype(jnp.float32)

        mb = m[:, s:s + _TB]
        ri = _fiota((_TB, _TB), 0)
        cj = _fiota((_TB, _TB), 1)
        tri = (ri < cj).astype(jnp.float32)
        mtri = mb * tri
        kin = keep[:, s:s + _TB]

        def _cond(st):
            return st[1]

        def _body(st):
            kb = st[0]
            sup = jnp.max(mtri * jnp.transpose(kb), axis=0, keepdims=True)
            kb2 = kin * (1.0 - jnp.minimum(sup, 1.0))
            return kb2, jnp.any(kb2 != kb)

        kb, _ = jax.lax.while_loop(_cond, _body, (kin, True))

        later = (colg > (s + _fiota((_TB, _NP), 0))).astype(jnp.float32)
        supg = jnp.max(m * later * jnp.transpose(kb), axis=0, keepdims=True)
        keep = keep * (1.0 - jnp.minimum(supg, 1.0))

    keep_ref[...] = jnp.broadcast_to(keep, (8, _NP))


def _run_nms(boxes_sorted):
    bp = jnp.zeros((8, _NP), jnp.float32)
    bp = bp.at[0:4, :_N].set(jnp.transpose(boxes_sorted))
    out = pl.pallas_call(
        _nms_kernel,
        out_shape=jax.ShapeDtypeStruct((8, _NP), jnp.float32),
    )(bp)
    return out[0, :_N] > 0.5


def _pool_kernel(boxes_ref, e_ref, out_ref):
    i = pl.program_id(0)
    cg = pl.program_id(1)
    bx = boxes_ref[:, pl.ds(i * _BOX_BLK, _BOX_BLK)]
    x1 = jnp.transpose(bx[0:1, :])
    y1 = jnp.transpose(bx[1:2, :])
    x2 = jnp.transpose(bx[2:3, :])
    y2 = jnp.transpose(bx[3:4, :])

    def _rep7(v):
        return jnp.concatenate([v] * 7, axis=0)

    def _ranges(lo, hi, hi_clip):
        a0 = jnp.clip(jnp.floor(lo / _STRIDE), 0.0, hi_clip - 1.0)
        a1 = jnp.clip(jnp.floor(hi / _STRIDE), 0.0, hi_clip)
        a1 = jnp.maximum(a1, a0 + 1.0)
        return _rep7(a0), _rep7(a1 - a0)

    r0, hh = _ranges(x1, x2, float(_H))
    c0, ww = _ranges(y1, y2, float(_W))

    rowi = _fiota((_ROWS_BLK, 1), 0)
    lc = jnp.floor((rowi + 0.5) / float(_BOX_BLK))
    ij = lc + 7.0 * cg.astype(jnp.float32)
    ci = jnp.floor((ij + 0.5) / 7.0)
    cjj = ij - ci * 7.0

    rs = r0 + jnp.floor((ci * hh + 0.5) / 7.0)
    re = r0 + jnp.floor(((ci + 1.0) * hh + 6.5) / 7.0)
    cs = c0 + jnp.floor((cjj * ww + 0.5) / 7.0)
    ce = c0 + jnp.floor(((cjj + 1.0) * ww + 6.5) / 7.0)

    col = _fiota((1, 640), 1)
    rcol = jnp.floor((col + 0.5) / 25.0)
    wcol = col - rcol * 25.0

    sel = ((rcol >= rs) & (rcol < re) & (wcol >= cs) & (wcol < ce))
    g = sel.astype(jnp.float32) / ((re - rs) * (ce - cs))
    out_ref[...] = jnp.dot(g, e_ref[...],
                           preferred_element_type=jnp.float32,
                           precision=jax.lax.Precision.HIGHEST)


def _run_pool(feature_map, boxes_k):
    e = jnp.transpose(feature_map, (1, 2, 0)).reshape(_H * _W, _C)
    e = jnp.pad(e, ((0, 640 - _H * _W), (0, 0)))
    bp = jnp.zeros((8, _KP), jnp.float32)
    bp = bp.at[0:4, :].set(jnp.transpose(boxes_k))
    n_blk = _KP // _BOX_BLK
    xp = pl.pallas_call(
        _pool_kernel,
        grid=(n_blk, 7),
        in_specs=[
            pl.BlockSpec((8, _KP), lambda i, cg: (0, 0)),
            pl.BlockSpec((640, _C), lambda i, cg: (0, 0)),
        ],
        out_specs=pl.BlockSpec((_ROWS_BLK, _C), lambda i, cg: (i * 7 + cg, 0)),
        out_shape=jax.ShapeDtypeStruct((_KP * 49, _C), jnp.float32),
        compiler_params=pltpu.CompilerParams(
            vmem_limit_bytes=64 * 1024 * 1024),
    )(bp, e)
    xp = xp.reshape(n_blk, 49, _BOX_BLK, _C)
    return jnp.transpose(xp, (0, 2, 3, 1)).reshape(_KP, _KDIM)


_KBLK = 896
_NBLK = 512


def _fc_kernel(x_ref, w_ref, b_ref, o_ref, acc_ref):
    k = pl.program_id(1)
    partial = jnp.dot(x_ref[:, pl.ds(k * _KBLK, _KBLK)], w_ref[...],
                      preferred_element_type=jnp.float32)

    @pl.when(k == 0)
    def _():
        acc_ref[...] = partial

    @pl.when(k > 0)
    def _():
        acc_ref[...] = acc_ref[...] + partial

    @pl.when(k == (_KDIM // _KBLK) - 1)
    def _():
        o_ref[...] = acc_ref[...] + b_ref[0:1, :]


def _run_fc(x, w_fc, b_fc):
    bb = jnp.broadcast_to(b_fc[None, :], (8, _FC))
    return pl.pallas_call(
        _fc_kernel,
        grid=(_FC // _NBLK, _KDIM // _KBLK),
        in_specs=[
            pl.BlockSpec((_KP, _KDIM), lambda n, k: (0, 0)),
            pl.BlockSpec((_KBLK, _NBLK), lambda n, k: (k, n)),
            pl.BlockSpec((8, _NBLK), lambda n, k: (0, n)),
        ],
        out_specs=pl.BlockSpec((_KP, _NBLK), lambda n, k: (0, n)),
        out_shape=jax.ShapeDtypeStruct((_KP, _FC), jnp.float32),
        scratch_shapes=[pltpu.VMEM((_KP, _NBLK), jnp.float32)],
        compiler_params=pltpu.CompilerParams(
            vmem_limit_bytes=64 * 1024 * 1024),
    )(x, w_fc, bb)


def _head_kernel(fc_ref, w_ref, b_ref, o_ref):
    t = jnp.dot(fc_ref[...], w_ref[...],
                preferred_element_type=jnp.float32) + b_ref[0:1, :]
    l = t[:, 0:4 * _NUM_CLASS]
    s = t[:, 4 * _NUM_CLASS:4 * _NUM_CLASS + _NUM_CLASS + 1]
    mx = jnp.max(s, axis=1, keepdims=True)
    idx = _fiota((_KP, _NUM_CLASS + 1), 1)
    mi = jnp.min(jnp.where(s == mx, idx, 1e9), axis=1, keepdims=True)
    pidx = _fiota((_KP, 4 * _NUM_CLASS), 1)
    full = _fiota((_KP, 128), 1)
    base = 4 * _NUM_CLASS + _NUM_CLASS + 1
    out = t
    for tshift in range(4):
        mask = (pidx == (mi + float(tshift))).astype(jnp.float32)
        col = jnp.sum(mask * l, axis=1, keepdims=True)
        dest = (full == float(base + tshift)).astype(jnp.float32)
        out = out * (1.0 - dest) + dest * col
    o_ref[...] = out


def _run_head(fc, w_l, b_l, w_s, b_s):
    wcat = jnp.concatenate([w_l, w_s], axis=1)
    wcat = jnp.pad(wcat, ((0, 0), (0, 128 - wcat.shape[1])))
    bcat = jnp.concatenate([b_l, b_s])
    bcat = jnp.pad(bcat, (0, 128 - bcat.shape[0]))
    bcat = jnp.broadcast_to(bcat[None, :], (8, 128))
    t = pl.pallas_call(
        _head_kernel,
        out_shape=jax.ShapeDtypeStruct((_KP, 128), jnp.float32),
    )(fc, wcat, bcat)
    base = 4 * _NUM_CLASS + _NUM_CLASS + 1
    l4 = t[:_K, base:base + 4]
    s = t[:_K, 4 * _NUM_CLASS:base]
    return l4, s


@functools.partial(jax.jit, static_argnums=())
def kernel(feature_map, proposals, scores, W_fc, b_fc, W_l, b_l, W_s, b_s):
    p = proposals[0]
    xy1 = p[:, :2] - p[:, 2:] / 2.0
    xy2 = xy1 + p[:, 2:]
    bbox = jnp.clip(jnp.concatenate([xy1, xy2], axis=-1), 0.0, 800.0)
    sm = jax.nn.softmax(scores, axis=2)[0, :, 1]
    order = jnp.argsort(-sm)
    bs = bbox[order]

    keep = _run_nms(bs)

    idx = jnp.arange(_N)
    pos = jnp.sort(jnp.where(keep, idx, _N))[:_K]
    pos = jnp.clip(pos, 0, _N - 1)
    keep_idx = order[pos]
    boxes_k = bbox[keep_idx]
    boxes_kp = jnp.pad(boxes_k, ((0, _KP - _K), (0, 0)))

    x = _run_pool(feature_map[0], boxes_kp)
    fc = _run_fc(x, W_fc, b_fc)
    l4, s = _run_head(fc, W_l, b_l, W_s, b_s)
    return (l4[None], s[None])
